# baseline (speedup 1.0000x reference)
import jax
import jax.numpy as jnp
from jax.experimental import pallas as pl
from jax.experimental.pallas import tpu as pltpu

_REP = 128
_NSEM = 8


def _body(pe_ref, o_hbm, stage_hbm, vm, sems):
    vm[...] = jnp.broadcast_to(pe_ref[...], vm.shape)
    pltpu.make_async_copy(vm, stage_hbm, sems.at[0]).start()
    pltpu.make_async_copy(vm, stage_hbm, sems.at[0]).wait()
    nchunks = o_hbm.shape[0] // _REP
    for j in range(nchunks):
        src = vm if j % 2 == 0 else stage_hbm
        pltpu.make_async_copy(
            src, o_hbm.at[pl.ds(j * _REP, _REP), :], sems.at[j % _NSEM]
        ).start()
    for j in range(nchunks):
        src = vm if j % 2 == 0 else stage_hbm
        pltpu.make_async_copy(
            src, o_hbm.at[pl.ds(j * _REP, _REP), :], sems.at[j % _NSEM]
        ).wait()


def kernel(x, pos_embed):
    batch = x.shape[0]
    max_len, d_model = pos_embed.shape
    row = max_len * d_model
    pe_flat = pos_embed.reshape(1, row)
    out, _ = pl.pallas_call(
        _body,
        in_specs=[pl.BlockSpec((1, row), lambda: (0, 0))],
        out_specs=[
            pl.BlockSpec(memory_space=pltpu.MemorySpace.HBM),
            pl.BlockSpec(memory_space=pltpu.MemorySpace.HBM),
        ],
        out_shape=[
            jax.ShapeDtypeStruct((batch, row), jnp.float32),
            jax.ShapeDtypeStruct((_REP, row), jnp.float32),
        ],
        scratch_shapes=[
            pltpu.VMEM((_REP, row), jnp.float32),
            pltpu.SemaphoreType.DMA((_NSEM,)),
        ],
    )(pe_flat)
    return out.reshape(batch, max_len, d_model)


# speedup vs baseline: 9.3331x; 9.3331x over previous
import functools

import jax
import jax.numpy as jnp
from jax import lax
from jax.experimental import pallas as pl
from jax.experimental.pallas import tpu as pltpu
from jax.experimental.pallas import tpu_sc as plsc

_REP = 8
_NW = 32
_CHUNKS = 4


def _sc_broadcast(pe_hbm, out_hbm, rep_v, sem):
    nc = 2
    wid = lax.axis_index("s") * nc + lax.axis_index("c")
    per_w = out_hbm.shape[0] // _NW
    base = wid * per_w
    for r in range(_REP):
        pltpu.sync_copy(pe_hbm, rep_v.at[pl.ds(r, 1)])
    copies = [
        pltpu.async_copy(rep_v, out_hbm.at[pl.ds(base + j * _REP, _REP)], sem)
        for j in range(per_w // _REP)
    ]
    for c in copies:
        c.wait()


def kernel(x, pos_embed):
    batch = x.shape[0]
    max_len, d_model = pos_embed.shape
    row = max_len * d_model
    pe_flat = pos_embed.reshape(1, row)
    mesh = plsc.VectorSubcoreMesh(core_axis_name="c", subcore_axis_name="s")
    chunk_rows = batch // _CHUNKS
    k = functools.partial(
        pl.kernel,
        mesh=mesh,
        out_type=jax.ShapeDtypeStruct((chunk_rows, row), jnp.float32),
        scratch_types=[
            pltpu.VMEM((_REP, row), jnp.float32),
            pltpu.SemaphoreType.DMA,
        ],
    )(_sc_broadcast)
    parts = [k(pe_flat) for _ in range(_CHUNKS)]
    out = jnp.concatenate(parts, axis=0)
    return out.reshape(batch, max_len, d_model)
